# baseline (speedup 1.0000x reference)
import functools

import jax
import jax.numpy as jnp
from jax.experimental import pallas as pl


def _body(x_ref, w_ref, b_ref, out_ref):
    s = jnp.sum(x_ref[...], axis=2)
    pooled = s * (1.0 / x_ref.shape[2])
    logits = jax.lax.dot_general(
        pooled, w_ref[...], (((1,), (1,)), ((), ())),
        preferred_element_type=jnp.float32,
    ) + b_ref[...]
    bb, e = logits.shape
    idx = jax.lax.broadcasted_iota(jnp.int32, (bb, e), 1)
    m1 = jnp.max(logits, axis=1, keepdims=True)
    i1 = jnp.min(jnp.where(logits == m1, idx, e), axis=1, keepdims=True)
    neg = jnp.where(idx == i1, -jnp.inf, logits)
    m2 = jnp.max(neg, axis=1, keepdims=True)
    i2 = jnp.min(jnp.where(neg == m2, idx, e), axis=1, keepdims=True)
    e2 = jnp.exp(m2 - m1)
    denom = 1.0 + e2
    w1 = 1.0 / denom
    w2 = e2 / denom
    out_ref[...] = jnp.where(idx == i1, w1, jnp.where(idx == i2, w2, 0.0))


@jax.jit
def kernel(x, W, b):
    B, C, H, Wd = x.shape
    E = W.shape[0]
    HW = H * Wd
    x3 = x.reshape(B, C, HW)
    b2 = b.reshape(1, E)
    BB = 8
    grid = (B // BB,)
    return pl.pallas_call(
        _body,
        grid=grid,
        in_specs=[
            pl.BlockSpec((BB, C, HW), lambda i: (i, 0, 0)),
            pl.BlockSpec((E, C), lambda i: (0, 0)),
            pl.BlockSpec((1, E), lambda i: (0, 0)),
        ],
        out_specs=pl.BlockSpec((BB, E), lambda i: (i, 0)),
        out_shape=jax.ShapeDtypeStruct((B, E), jnp.float32),
    )(x3, W, b2)


# speedup vs baseline: 4.1495x; 4.1495x over previous
import functools

import jax
import jax.numpy as jnp
from jax.experimental import pallas as pl


def _body(x_ref, w_ref, b_ref, out_ref):
    s = jnp.sum(x_ref[...], axis=1)
    pooled = s * (1.0 / x_ref.shape[1])
    logits = jax.lax.dot_general(
        pooled, w_ref[...], (((1,), (1,)), ((), ())),
        preferred_element_type=jnp.float32,
    ) + b_ref[...]
    bb, e = logits.shape
    idx = jax.lax.broadcasted_iota(jnp.int32, (bb, e), 1)
    m1 = jnp.max(logits, axis=1, keepdims=True)
    i1 = jnp.min(jnp.where(logits == m1, idx, e), axis=1, keepdims=True)
    neg = jnp.where(idx == i1, -jnp.inf, logits)
    m2 = jnp.max(neg, axis=1, keepdims=True)
    i2 = jnp.min(jnp.where(neg == m2, idx, e), axis=1, keepdims=True)
    e2 = jnp.exp(m2 - m1)
    denom = 1.0 + e2
    w1 = 1.0 / denom
    w2 = e2 / denom
    out_ref[...] = jnp.where(idx == i1, w1, jnp.where(idx == i2, w2, 0.0))


@jax.jit
def kernel(x, W, b):
    B, C, H, Wd = x.shape
    E = W.shape[0]
    HW = H * Wd
    x3 = jnp.transpose(x, (0, 2, 3, 1)).reshape(B, HW, C)
    b2 = b.reshape(1, E)
    BB = 8
    grid = (B // BB,)
    return pl.pallas_call(
        _body,
        grid=grid,
        in_specs=[
            pl.BlockSpec((BB, HW, C), lambda i: (i, 0, 0)),
            pl.BlockSpec((E, C), lambda i: (0, 0)),
            pl.BlockSpec((1, E), lambda i: (0, 0)),
        ],
        out_specs=pl.BlockSpec((BB, E), lambda i: (i, 0)),
        out_shape=jax.ShapeDtypeStruct((B, E), jnp.float32),
    )(x3, W, b2)
